# baseline (speedup 1.0000x reference)
import functools

import jax
import jax.numpy as jnp
from jax import lax
from jax.experimental import pallas as pl
from jax.experimental.pallas import tpu as pltpu
from jax.experimental.pallas import tpu_sc as plsc

_NUM_CORES = 2
_NUM_SUBCORES = 16
_NW = _NUM_CORES * _NUM_SUBCORES

_LANES = 16
_ROWS = 256
_TCOLS = _ROWS // 128
_OBUF = _TCOLS * 1024


@functools.partial(jax.jit, static_argnums=(2, 3))
def _gather_tiled(idx, table4, n, d):
    per_w = n // _NW
    n_chunks = per_w // _ROWS
    cols_per_w = per_w // 128
    d4 = 4 * d

    @functools.partial(
        pl.kernel,
        out_type=jax.ShapeDtypeStruct((n * d,), jnp.float32),
        mesh=plsc.VectorSubcoreMesh(core_axis_name="c", subcore_axis_name="s"),
        scratch_types=[
            pltpu.VMEM((per_w,), jnp.int32),
            pltpu.VMEM((_ROWS,), jnp.int32),
            pltpu.VMEM((_ROWS,), jnp.int32),
            pltpu.VMEM((_ROWS, d4), jnp.float32),
            pltpu.VMEM((_ROWS, d4), jnp.float32),
            pltpu.VMEM((4 * _OBUF,), jnp.float32),
            pltpu.VMEM((4 * _OBUF,), jnp.float32),
            pltpu.SemaphoreType.DMA((2,)),
            pltpu.SemaphoreType.DMA((2,)),
        ],
        compiler_params=pltpu.CompilerParams(
            use_tc_tiling_on_sc=True, needs_layout_passes=False),
    )
    def _impl(idx_hbm, table_hbm, out_hbm, idx_v, idxq0, idxq1, rbuf0, rbuf1,
              tbuf0, tbuf1, gsem, wsem):
        idxq = (idxq0, idxq1)
        rbuf = (rbuf0, rbuf1)
        tbuf = (tbuf0, tbuf1)
        wid = lax.axis_index("s") * _NUM_CORES + lax.axis_index("c")
        base = wid * per_w
        tcol0 = wid * cols_per_w

        pltpu.sync_copy(idx_hbm.at[pl.ds(base, per_w)], idx_v)

        iota = lax.iota(jnp.int32, _LANES)
        fpat = [lax.rem(iota + k, _LANES) for k in range(_LANES)]
        dpat = [(fp // 8) * _OBUF + lax.rem(fp, 8) * 128 + iota for fp in fpat]

        def prep_idxq(c, b):
            def u_body(u, carry):
                v = idx_v[pl.ds(c * _ROWS + u * _LANES, _LANES)]
                idxq[b][pl.ds(u * _LANES, _LANES)] = v >> 2
                return carry

            lax.fori_loop(0, _ROWS // _LANES, u_body, 0)

        def gather_of(c, b):
            return pltpu.make_async_copy(
                table_hbm.at[idxq[b]], rbuf[b], gsem.at[b])

        def write_of(c, b, ft):
            t0 = tcol0 + c * _TCOLS
            l0 = t0 // 128
            bt0 = lax.rem(t0, 128)
            off = ((l0 * 4 + ft) * 128 + bt0) * 1024
            return pltpu.make_async_copy(
                tbuf[b].at[pl.ds(ft * _OBUF, _OBUF)],
                out_hbm.at[pl.ds(off, _OBUF)],
                wsem.at[b],
            )

        def do_sub(c, b):
            gather_of(c, b).wait()

            @pl.when(c >= 2)
            def _():
                for ft in range(4):
                    write_of(c - 2, b, ft).wait()

            rb = rbuf[b]
            tflat = tbuf[b]

            def g_body(g, carry):
                r0 = g * _LANES
                rows = iota + r0
                ovec = (idx_v[pl.ds(c * _ROWS + r0, _LANES)] & 3) << 5
                sbase0 = (r0 // 128) * 1024 + lax.rem(r0, 128)
                for f0 in (0, 16):
                    cbase = ovec + f0
                    sbase = sbase0 + (f0 // 8) * _OBUF
                    for k in range(_LANES):
                        v = plsc.load_gather(rb, [rows, cbase + fpat[k]])
                        plsc.store_scatter(tflat, [dpat[k] + sbase], v)
                return carry

            lax.fori_loop(0, _ROWS // _LANES, g_body, 0)

            for ft in range(4):
                write_of(c, b, ft).start()

            @pl.when(c + 2 < n_chunks)
            def _():
                prep_idxq(c + 2, b)
                gather_of(c + 2, b).start()

        prep_idxq(0, 0)
        gather_of(0, 0).start()
        prep_idxq(1, 1)
        gather_of(1, 1).start()

        def body(k, carry):
            do_sub(2 * k, 0)
            do_sub(2 * k + 1, 1)
            return carry

        lax.fori_loop(0, n_chunks // 2, body, 0)

        for c in (n_chunks - 2, n_chunks - 1):
            for ft in range(4):
                write_of(c, c % 2, ft).wait()

    return _impl(idx, table4)


def kernel(x, table):
    b, l = x.shape
    v, d = table.shape
    n = b * l
    out = _gather_tiled(x.T.reshape(n), table.reshape(v // 4, 4 * d), n, d)
    o5 = out.reshape(l, 4, b // 128, 8, 128)
    return o5.transpose((2, 4, 0, 1, 3)).reshape(b, l, d)


# speedup vs baseline: 1.5219x; 1.5219x over previous
import functools

import jax
import jax.numpy as jnp
from jax import lax
from jax.experimental import pallas as pl
from jax.experimental.pallas import tpu as pltpu
from jax.experimental.pallas import tpu_sc as plsc

_NUM_CORES = 2
_NUM_SUBCORES = 16
_NW = _NUM_CORES * _NUM_SUBCORES

_LANES = 16
_ROWS = 256
_TCOLS = _ROWS // 128
_OBUF = _TCOLS * 1024

_TILE = 128
_NTILE = 7812
_PERW = _NTILE // _NW


def _diag_patterns():
    iota = lax.iota(jnp.int32, _LANES)
    fpat = [lax.rem(iota + k, _LANES) for k in range(_LANES)]
    return iota, fpat


@functools.partial(jax.jit, static_argnums=(2, 3))
def _transpose_table(table_t, table_tail_t, v, d):
    n_extra = _NTILE - _PERW * _NW
    tail = v - _NTILE * _TILE

    @functools.partial(
        pl.kernel,
        out_type=jax.ShapeDtypeStruct((v * d,), jnp.float32),
        mesh=plsc.VectorSubcoreMesh(core_axis_name="c", subcore_axis_name="s"),
        scratch_types=[
            pltpu.VMEM((d, _TILE), jnp.float32),
            pltpu.VMEM((d, _TILE), jnp.float32),
            pltpu.VMEM((_TILE * d,), jnp.float32),
            pltpu.VMEM((_TILE * d,), jnp.float32),
            pltpu.VMEM((d, tail), jnp.float32),
            pltpu.VMEM((tail * d,), jnp.float32),
            pltpu.SemaphoreType.DMA((2,)),
            pltpu.SemaphoreType.DMA((2,)),
        ],
        compiler_params=pltpu.CompilerParams(
            use_tc_tiling_on_sc=True, needs_layout_passes=False),
    )
    def _impl(tt_hbm, ttail_hbm, out_hbm, vb0, vb1, ob0, ob1, vtail, otail,
              gsem, wsem):
        vb = (vb0, vb1)
        ob = (ob0, ob1)
        wid = lax.axis_index("s") * _NUM_CORES + lax.axis_index("c")
        base = wid * _PERW

        iota, fpat = _diag_patterns()
        spat = [iota * d + fp for fp in fpat]

        def transpose_block(src, dst, n_groups):
            def g_body(g, carry):
                v0 = g * _LANES
                cols = iota + v0
                for f0 in (0, 16):
                    for k in range(_LANES):
                        val = plsc.load_gather(src, [fpat[k] + f0, cols])
                        plsc.store_scatter(dst, [spat[k] + (v0 * d + f0)], val)
                return carry

            lax.fori_loop(0, n_groups, g_body, 0)

        def read_of(t, b):
            return pltpu.make_async_copy(
                tt_hbm.at[:, pl.ds(t * _TILE, _TILE)], vb[b], gsem.at[b])

        def write_of(t, b):
            return pltpu.make_async_copy(
                ob[b], out_hbm.at[pl.ds(t * _TILE * d, _TILE * d)], wsem.at[b])

        def do_sub(t, b, first, prefetch_t):
            read_of(t, b).wait()

            @pl.when(jnp.logical_not(first))
            def _():
                write_of(t, b).wait()

            transpose_block(vb[b], ob[b], _TILE // _LANES)
            write_of(t, b).start()

            @pl.when(prefetch_t < base + _PERW)
            def _():
                read_of(prefetch_t, b).start()

        @pl.when(wid == 0)
        def _():
            pltpu.sync_copy(ttail_hbm, vtail)
            transpose_block(vtail, otail, tail // _LANES)
            pltpu.sync_copy(otail, out_hbm.at[pl.ds(_NTILE * _TILE * d, tail * d)])

        read_of(base, 0).start()
        read_of(base + 1, 1).start()

        def body(k, carry):
            do_sub(base + 2 * k, 0, k == 0, base + 2 * k + 2)
            do_sub(base + 2 * k + 1, 1, k == 0, base + 2 * k + 3)
            return carry

        lax.fori_loop(0, _PERW // 2, body, 0)

        @pl.when(wid < n_extra)
        def _():
            t = _PERW * _NW + wid
            write_of(t, 0).wait()
            read_of(t, 0).start()
            read_of(t, 0).wait()
            transpose_block(vb[0], ob[0], _TILE // _LANES)
            write_of(t, 0).start()
            write_of(t, 0).wait()

        @pl.when(wid >= n_extra)
        def _():
            write_of(0, 0).wait()

        write_of(0, 1).wait()

    return _impl(table_t, table_tail_t)


@functools.partial(jax.jit, static_argnums=(2, 3))
def _gather_tiled(idx, table, n, d):
    per_w = n // _NW
    n_chunks = per_w // _ROWS
    cols_per_w = per_w // 128

    @functools.partial(
        pl.kernel,
        out_type=jax.ShapeDtypeStruct((n * d,), jnp.float32),
        mesh=plsc.VectorSubcoreMesh(core_axis_name="c", subcore_axis_name="s"),
        scratch_types=[
            pltpu.VMEM((per_w,), jnp.int32),
            pltpu.VMEM((2, _ROWS, d), jnp.float32),
            pltpu.VMEM((2, 4 * _OBUF), jnp.float32),
            pltpu.SemaphoreType.DMA((2,)),
            pltpu.SemaphoreType.DMA((2,)),
        ],
        compiler_params=pltpu.CompilerParams(
            use_tc_tiling_on_sc=False, needs_layout_passes=False),
    )
    def _impl(idx_hbm, table_hbm, out_hbm, idx_v, rbuf, tbuf, gsem, wsem):
        wid = lax.axis_index("s") * _NUM_CORES + lax.axis_index("c")
        base = wid * per_w
        tcol0 = wid * cols_per_w

        pltpu.sync_copy(idx_hbm.at[pl.ds(base, per_w)], idx_v)

        iota, fpat = _diag_patterns()
        dpat = [(fp // 8) * _OBUF + lax.rem(fp, 8) * 128 + iota for fp in fpat]

        def gather_of(c, b):
            return pltpu.make_async_copy(
                table_hbm.at[idx_v.at[pl.ds(c * _ROWS, _ROWS)]],
                rbuf.at[b],
                gsem.at[b],
            )

        def write_of(c, b, ft):
            t0 = tcol0 + c * _TCOLS
            l0 = t0 // 128
            bt0 = lax.rem(t0, 128)
            off = ((l0 * 4 + ft) * 128 + bt0) * 1024
            return pltpu.make_async_copy(
                tbuf.at[b, pl.ds(ft * _OBUF, _OBUF)],
                out_hbm.at[pl.ds(off, _OBUF)],
                wsem.at[b],
            )

        def do_sub(c, b):
            gather_of(c, b).wait()

            @pl.when(c >= 2)
            def _():
                for ft in range(4):
                    write_of(c - 2, b, ft).wait()

            rb = rbuf.at[b]
            tflat = tbuf.at[b]

            def g_body(g, carry):
                r0 = g * _LANES
                rows = iota + r0
                sbase0 = (r0 // 128) * 1024 + lax.rem(r0, 128)
                for f0 in (0, 16):
                    sbase = sbase0 + (f0 // 8) * _OBUF
                    for k in range(_LANES):
                        v = plsc.load_gather(rb, [rows, fpat[k] + f0])
                        plsc.store_scatter(tflat, [dpat[k] + sbase], v)
                return carry

            lax.fori_loop(0, _ROWS // _LANES, g_body, 0)

            for ft in range(4):
                write_of(c, b, ft).start()

            @pl.when(c + 2 < n_chunks)
            def _():
                gather_of(c + 2, b).start()

        gather_of(0, 0).start()
        gather_of(1, 1).start()

        def body(k, carry):
            do_sub(2 * k, 0)
            do_sub(2 * k + 1, 1)
            return carry

        lax.fori_loop(0, n_chunks // 2, body, 0)

        for c in (n_chunks - 2, n_chunks - 1):
            for ft in range(4):
                write_of(c, c % 2, ft).wait()

    return _impl(idx, table)


def kernel(x, table):
    b, l = x.shape
    v, d = table.shape
    n = b * l
    table_lin = _transpose_table(
        table.T, table[_NTILE * _TILE:].T, v, d).reshape(v, d)
    out = _gather_tiled(x.T.reshape(n), table_lin, n, d)
    o5 = out.reshape(l, 4, b // 128, 8, 128)
    return o5.transpose((2, 4, 0, 1, 3)).reshape(b, l, d)
